# baseline (speedup 1.0000x reference)
import functools

import jax
import jax.numpy as jnp
from jax import lax
from jax.experimental import pallas as pl
from jax.experimental.pallas import tpu as pltpu
from jax.experimental.pallas import tpu_sc as plsc

RESO = 128
CHUNK = 4000
LANES = 16
UNROLL = 4
TAB_ROWS = 16


def _table_body(off_ref, x_ref, y_ref, z_ref, tab_ref):
    f32 = jnp.float32
    row = lax.broadcasted_iota(jnp.int32, (RESO, RESO), 0)
    col = lax.broadcasted_iota(jnp.int32, (RESO, RESO), 1)
    cum = (row <= col).astype(f32)
    shf = (row == col - 1).astype(f32)
    kpos = (lax.broadcasted_iota(jnp.int32, (1, RESO), 1) > 0).astype(f32)
    oneh0 = 1.0 - kpos

    def dot(a, b):
        return jnp.dot(a, b, preferred_element_type=f32)

    for t, l_ref in enumerate((x_ref, y_ref, z_ref)):
        l = l_ref[...]
        off_t = off_ref[t]
        l_m1 = dot(l, shf)
        d = 2.0 * (l_m1 + l * kpos) + off_t * oneh0
        a = dot(d, cum)
        a_m1 = dot(a, shf)
        e = 3.0 * l_m1 + l * kpos + 2.0 * a_m1
        a6 = dot(e, cum)
        if t == 0:
            a6 = a6 + off_ref[3]
        tab_ref[pl.ds(t, 1), :] = l
        tab_ref[pl.ds(3 + t, 1), :] = a
        tab_ref[pl.ds(6 + t, 1), :] = a6
    tab_ref[pl.ds(9, TAB_ROWS - 9), :] = jnp.zeros(
        (TAB_ROWS - 9, RESO), f32)


def _build_tables(offset, x_l, y_l, z_l):
    return pl.pallas_call(
        _table_body,
        out_shape=jax.ShapeDtypeStruct((TAB_ROWS, RESO), jnp.float32),
        in_specs=[
            pl.BlockSpec(memory_space=pltpu.SMEM),
            pl.BlockSpec(),
            pl.BlockSpec(),
            pl.BlockSpec(),
        ],
        out_specs=pl.BlockSpec(),
    )(offset, x_l.reshape(1, RESO), y_l.reshape(1, RESO),
      z_l.reshape(1, RESO))


def _rsqrt(x):
    i = lax.bitcast_convert_type(x, jnp.int32)
    y = lax.bitcast_convert_type(jnp.int32(0x5F3759DF) - (i >> 1), jnp.float32)
    xh = 0.5 * x
    for _ in range(2):
        y = y * (1.5 - xh * y * y)
    return y


def _full(v):
    return jnp.full((LANES,), v, jnp.int32)


def _sc_phase(sdf_mode, n, mesh, num_cores, num_workers, n_chunks, n_rounds):
    n_out = 1 if sdf_mode else 3

    @functools.partial(
        pl.kernel,
        out_type=(jax.ShapeDtypeStruct((n,), jnp.float32) if sdf_mode
                  else jax.ShapeDtypeStruct((4 * n,), jnp.float32),),
        mesh=mesh,
        compiler_params=pltpu.CompilerParams(needs_layout_passes=False),
        scratch_types=(
            [pltpu.VMEM((TAB_ROWS * RESO,), jnp.float32)]
            + [pltpu.VMEM((CHUNK,), jnp.int32) if k % 2 == 0 else
               pltpu.VMEM((4 * CHUNK,), jnp.float32) for k in range(4)]
            + [pltpu.VMEM((CHUNK,), jnp.float32) if sdf_mode else
               pltpu.VMEM((4 * CHUNK,), jnp.float32) for _ in range(2)]
            + [pltpu.SemaphoreType.DMA for _ in range(4)]
        ),
    )
    def run(pts_h, idx_h, tab_hbm, *rest):
        out_h = rest[0]
        tab = rest[1]
        bufs = rest[2:6]
        obufs = rest[6:8]
        in_sems = rest[-4:-2]
        out_sems = rest[-2:]
        pltpu.sync_copy(tab_hbm, tab)
        in_bufs = (bufs[0:2], bufs[2:4])
        out_bufs = (obufs[0], obufs[1])

        wid = lax.axis_index("s") * num_cores + lax.axis_index("c")

        def chunk_j(r):
            return wid + r * num_workers

        def in_srcs(j):
            return ((idx_h.at[pl.ds(j * CHUNK, CHUNK)],),
                    (pts_h.at[pl.ds(j * 4 * CHUNK, 4 * CHUNK)],))

        def out_pairs(j, b):
            if sdf_mode:
                return ((out_bufs[b], out_h.at[pl.ds(j * CHUNK, CHUNK)]),)
            return ((out_bufs[b], out_h.at[pl.ds(j * 4 * CHUNK, 4 * CHUNK)]),)

        def fire_ins(r, b):
            j = chunk_j(r)

            @pl.when(j < n_chunks)
            def _():
                (i_src,), (p_src,) = in_srcs(j)
                pltpu.async_copy(i_src, in_bufs[b][0], in_sems[b])
                pltpu.async_copy(p_src, in_bufs[b][1], in_sems[b])

        def wait_ins(r, b):
            j = chunk_j(r)

            @pl.when(j < n_chunks)
            def _():
                (i_src,), (p_src,) = in_srcs(j)
                pltpu.make_async_copy(i_src, in_bufs[b][0], in_sems[b]).wait()
                pltpu.make_async_copy(p_src, in_bufs[b][1], in_sems[b]).wait()

        def wait_outs(r, b):
            j = chunk_j(r)

            @pl.when((j >= 0) & (j < n_chunks))
            def _():
                for s_ref, d_ref in out_pairs(j, b):
                    pltpu.make_async_copy(s_ref, d_ref, out_sems[b]).wait()

        def compute(r, b):
            j = chunk_j(r)

            @pl.when(j < n_chunks)
            def _():
                bi, bp = in_bufs[b]
                ob = out_bufs[b]
                iota16 = lax.broadcasted_iota(jnp.int32, (LANES,), 0)

                def decode(s):
                    idxv = bi[pl.ds(s, 16)]
                    xv = idxv & (RESO - 1)
                    yv = ((idxv >> 7) & (RESO - 1)) + 1 * RESO
                    zv = (idxv >> 14) + 2 * RESO
                    rows4 = 4 * s + 4 * iota16
                    return (rows4, xv, yv, zv,
                            plsc.load_gather(bp, [rows4]),
                            plsc.load_gather(bp, [rows4 + 1]),
                            plsc.load_gather(bp, [rows4 + 2]))

                def sdf_vec(s):
                    rows4, xv, yv, zv, px, py, pz = decode(s)
                    a0 = plsc.load_gather(tab, [xv]) * px
                    a1 = plsc.load_gather(tab, [yv]) * py
                    a2 = plsc.load_gather(tab, [zv]) * pz
                    a3 = plsc.load_gather(tab, [xv + 3 * RESO])
                    a4 = plsc.load_gather(tab, [yv + 3 * RESO])
                    a5 = plsc.load_gather(tab, [zv + 3 * RESO])
                    a6 = (plsc.load_gather(tab, [xv + 6 * RESO])
                          + plsc.load_gather(tab, [yv + 6 * RESO])
                          + plsc.load_gather(tab, [zv + 6 * RESO]))
                    num = ((a0 + a3) * px + (a1 + a4) * py
                           + (a2 + a5) * pz + a6)
                    u = 2.0 * a0 + a3
                    v = 2.0 * a1 + a4
                    w = 2.0 * a2 + a5
                    ob[pl.ds(s, 16)] = (num * _rsqrt(u * u + v * v + w * w)
                                        * (1.0 / RESO))

                def ren_vec(s):
                    rows4, xv, yv, zv, px, py, pz = decode(s)
                    g0 = 2.0 * plsc.load_gather(tab, [xv]) * px \
                        + plsc.load_gather(tab, [xv + 3 * RESO])
                    g1 = 2.0 * plsc.load_gather(tab, [yv]) * py \
                        + plsc.load_gather(tab, [yv + 3 * RESO])
                    g2 = 2.0 * plsc.load_gather(tab, [zv]) * pz \
                        + plsc.load_gather(tab, [zv + 3 * RESO])
                    rs = _rsqrt(jnp.maximum(g0 * g0 + g1 * g1 + g2 * g2,
                                            1e-24))
                    plsc.store_scatter(ob, [rows4], g0 * rs)
                    plsc.store_scatter(ob, [rows4 + 1], g1 * rs)
                    plsc.store_scatter(ob, [rows4 + 2], g2 * rs)

                body = sdf_vec if sdf_mode else ren_vec
                plsc.parallel_loop(0, CHUNK, LANES, unroll=UNROLL)(body)
                for o, d in out_pairs(j, b):
                    pltpu.async_copy(o, d, out_sems[b])

        def step(r, b):
            fire_ins(r + 1, 1 - b)
            wait_ins(r, b)
            wait_outs(r - 2, b)
            compute(r, b)

        fire_ins(jnp.int32(0), 0)

        def pair_body(rr, carry_):
            r = 2 * rr
            step(r, 0)
            step(r + 1, 1)
            return carry_

        lax.fori_loop(0, n_rounds // 2, pair_body, 0)
        wait_outs(jnp.int32(n_rounds - 2), 0)
        wait_outs(jnp.int32(n_rounds - 1), 1)

    return run


def kernel(renderPointList, renderIndexList, sdfPointList, sdfIndexList,
           xLayer, yLayer, zLayer, offset):
    n = sdfPointList.shape[0]
    assert n % CHUNK == 0
    n_chunks = n // CHUNK

    mesh = plsc.VectorSubcoreMesh(core_axis_name="c", subcore_axis_name="s")
    num_cores = mesh.num_cores
    num_workers = num_cores * mesh.num_subcores
    n_rounds = -(-n_chunks // num_workers)
    n_rounds += n_rounds % 2

    run_sdf = _sc_phase(True, n, mesh, num_cores, num_workers,
                        n_chunks, n_rounds)
    run_ren = _sc_phase(False, n, mesh, num_cores, num_workers,
                        n_chunks, n_rounds)

    tab = _build_tables(offset, xLayer, yLayer, zLayer).reshape(-1)
    s4 = jnp.pad(sdfPointList, ((0, 0), (0, 1))).reshape(-1)
    r4 = jnp.pad(renderPointList, ((0, 0), (0, 1))).reshape(-1)
    (sdf_list,) = run_sdf(s4, sdfIndexList, tab)
    (nrm4,) = run_ren(r4, renderIndexList, tab)
    return (sdf_list, nrm4.reshape(n, 4)[:, :3])


# speedup vs baseline: 44.9676x; 44.9676x over previous
import functools

import jax
import jax.numpy as jnp
from jax import lax
from jax.experimental import pallas as pl
from jax.experimental.pallas import tpu as pltpu
from jax.experimental.pallas import tpu_sc as plsc

RESO = 128
CHUNK = 4000
LANES = 16
UNROLL = 4
TAB_ROWS = 16


def _table_body(off_ref, x_ref, y_ref, z_ref, tab_ref):
    f32 = jnp.float32
    row = lax.broadcasted_iota(jnp.int32, (RESO, RESO), 0)
    col = lax.broadcasted_iota(jnp.int32, (RESO, RESO), 1)
    cum = (row <= col).astype(f32)
    shf = (row == col - 1).astype(f32)
    kpos = (lax.broadcasted_iota(jnp.int32, (1, RESO), 1) > 0).astype(f32)
    oneh0 = 1.0 - kpos

    def dot(a, b):
        return jnp.dot(a, b, preferred_element_type=f32)

    for t, l_ref in enumerate((x_ref, y_ref, z_ref)):
        l = l_ref[...]
        off_t = off_ref[t]
        l_m1 = dot(l, shf)
        d = 2.0 * (l_m1 + l * kpos) + off_t * oneh0
        a = dot(d, cum)
        a_m1 = dot(a, shf)
        e = 3.0 * l_m1 + l * kpos + 2.0 * a_m1
        a6 = dot(e, cum)
        if t == 0:
            a6 = a6 + off_ref[3]
        tab_ref[pl.ds(t, 1), :] = l
        tab_ref[pl.ds(3 + t, 1), :] = a
        tab_ref[pl.ds(6 + t, 1), :] = a6
    tab_ref[pl.ds(9, TAB_ROWS - 9), :] = jnp.zeros(
        (TAB_ROWS - 9, RESO), f32)


def _build_tables(offset, x_l, y_l, z_l):
    return pl.pallas_call(
        _table_body,
        out_shape=jax.ShapeDtypeStruct((TAB_ROWS, RESO), jnp.float32),
        in_specs=[
            pl.BlockSpec(memory_space=pltpu.SMEM),
            pl.BlockSpec(),
            pl.BlockSpec(),
            pl.BlockSpec(),
        ],
        out_specs=pl.BlockSpec(),
    )(offset, x_l.reshape(1, RESO), y_l.reshape(1, RESO),
      z_l.reshape(1, RESO))


def _rsqrt(x):
    i = lax.bitcast_convert_type(x, jnp.int32)
    y = lax.bitcast_convert_type(jnp.int32(0x5F3759DF) - (i >> 1), jnp.float32)
    xh = 0.5 * x
    for _ in range(2):
        y = y * (1.5 - xh * y * y)
    return y


def _full(v):
    return jnp.full((LANES,), v, jnp.int32)


def _sc_phase(sdf_mode, n, mesh, num_cores, num_workers, n_chunks, n_rounds):
    n_out = 1 if sdf_mode else 3

    @functools.partial(
        pl.kernel,
        out_type=tuple(jax.ShapeDtypeStruct((n,), jnp.float32)
                       for _ in range(n_out)),
        mesh=mesh,
        compiler_params=pltpu.CompilerParams(needs_layout_passes=False),
        scratch_types=(
            [pltpu.VMEM((TAB_ROWS * RESO,), jnp.float32)]
            + [pltpu.VMEM((CHUNK,), jnp.int32) if k % 4 == 0 else
               pltpu.VMEM((CHUNK,), jnp.float32) for k in range(8)]
            + [pltpu.VMEM((CHUNK,), jnp.float32) for _ in range(2 * n_out)]
            + [pltpu.SemaphoreType.DMA for _ in range(4)]
        ),
    )
    def run(px_h, py_h, pz_h, idx_h, tab_hbm, *rest):
        outs = rest[:n_out]
        tab = rest[n_out]
        bufs = rest[n_out + 1:n_out + 9]
        obufs = rest[n_out + 9:n_out + 9 + 2 * n_out]
        in_sems = rest[-4:-2]
        out_sems = rest[-2:]
        pltpu.sync_copy(tab_hbm, tab)
        in_bufs = (bufs[0:4], bufs[4:8])
        out_bufs = (obufs[0:n_out], obufs[n_out:])

        wid = lax.axis_index("s") * num_cores + lax.axis_index("c")

        def chunk_j(r):
            return wid + r * num_workers

        def in_srcs(sl):
            return (idx_h.at[sl], px_h.at[sl], py_h.at[sl], pz_h.at[sl])

        def out_pairs(j, b):
            sl = pl.ds(j * CHUNK, CHUNK)
            return tuple((o, outs[k].at[sl]) for k, o in enumerate(out_bufs[b]))

        def fire_ins(r, b):
            j = chunk_j(r)

            @pl.when(j < n_chunks)
            def _():
                sl = pl.ds(j * CHUNK, CHUNK)
                for s_ref, d_ref in zip(in_srcs(sl), in_bufs[b]):
                    pltpu.async_copy(s_ref, d_ref, in_sems[b])

        def wait_ins(r, b):
            j = chunk_j(r)

            @pl.when(j < n_chunks)
            def _():
                sl = pl.ds(j * CHUNK, CHUNK)
                for s_ref, d_ref in zip(in_srcs(sl), in_bufs[b]):
                    pltpu.make_async_copy(s_ref, d_ref, in_sems[b]).wait()

        def wait_outs(r, b):
            j = chunk_j(r)

            @pl.when((j >= 0) & (j < n_chunks))
            def _():
                for s_ref, d_ref in out_pairs(j, b):
                    pltpu.make_async_copy(s_ref, d_ref, out_sems[b]).wait()

        def compute(r, b):
            j = chunk_j(r)

            @pl.when(j < n_chunks)
            def _():
                sl = pl.ds(j * CHUNK, CHUNK)
                bi, bx, by, bz = in_bufs[b]
                ob = out_bufs[b]

                def decode(s):
                    idxv = bi[pl.ds(s, 16)]
                    xv = idxv & (RESO - 1)
                    yv = ((idxv >> 7) & (RESO - 1)) + 1 * RESO
                    zv = (idxv >> 14) + 2 * RESO
                    return (xv, yv, zv, bx[pl.ds(s, 16)],
                            by[pl.ds(s, 16)], bz[pl.ds(s, 16)])

                def sdf_vec(s):
                    xv, yv, zv, px, py, pz = decode(s)
                    a0 = plsc.load_gather(tab, [xv]) * px
                    a1 = plsc.load_gather(tab, [yv]) * py
                    a2 = plsc.load_gather(tab, [zv]) * pz
                    a3 = plsc.load_gather(tab, [xv + 3 * RESO])
                    a4 = plsc.load_gather(tab, [yv + 3 * RESO])
                    a5 = plsc.load_gather(tab, [zv + 3 * RESO])
                    a6 = (plsc.load_gather(tab, [xv + 6 * RESO])
                          + plsc.load_gather(tab, [yv + 6 * RESO])
                          + plsc.load_gather(tab, [zv + 6 * RESO]))
                    num = ((a0 + a3) * px + (a1 + a4) * py
                           + (a2 + a5) * pz + a6)
                    u = 2.0 * a0 + a3
                    v = 2.0 * a1 + a4
                    w = 2.0 * a2 + a5
                    ob[0][pl.ds(s, 16)] = (num * _rsqrt(u * u + v * v + w * w)
                                           * (1.0 / RESO))

                def ren_vec(s):
                    xv, yv, zv, px, py, pz = decode(s)
                    g0 = 2.0 * plsc.load_gather(tab, [xv]) * px \
                        + plsc.load_gather(tab, [xv + 3 * RESO])
                    g1 = 2.0 * plsc.load_gather(tab, [yv]) * py \
                        + plsc.load_gather(tab, [yv + 3 * RESO])
                    g2 = 2.0 * plsc.load_gather(tab, [zv]) * pz \
                        + plsc.load_gather(tab, [zv + 3 * RESO])
                    rs = _rsqrt(jnp.maximum(g0 * g0 + g1 * g1 + g2 * g2,
                                            1e-24))
                    ob[0][pl.ds(s, 16)] = g0 * rs
                    ob[1][pl.ds(s, 16)] = g1 * rs
                    ob[2][pl.ds(s, 16)] = g2 * rs

                body = sdf_vec if sdf_mode else ren_vec
                plsc.parallel_loop(0, CHUNK, LANES, unroll=UNROLL)(body)
                for o, d in out_pairs(j, b):
                    pltpu.async_copy(o, d, out_sems[b])

        def step(r, b):
            fire_ins(r + 1, 1 - b)
            wait_ins(r, b)
            wait_outs(r - 2, b)
            compute(r, b)

        fire_ins(jnp.int32(0), 0)

        def pair_body(rr, carry_):
            r = 2 * rr
            step(r, 0)
            step(r + 1, 1)
            return carry_

        lax.fori_loop(0, n_rounds // 2, pair_body, 0)
        wait_outs(jnp.int32(n_rounds - 2), 0)
        wait_outs(jnp.int32(n_rounds - 1), 1)

    return run


def kernel(renderPointList, renderIndexList, sdfPointList, sdfIndexList,
           xLayer, yLayer, zLayer, offset):
    n = sdfPointList.shape[0]
    assert n % CHUNK == 0
    n_chunks = n // CHUNK

    mesh = plsc.VectorSubcoreMesh(core_axis_name="c", subcore_axis_name="s")
    num_cores = mesh.num_cores
    num_workers = num_cores * mesh.num_subcores
    n_rounds = -(-n_chunks // num_workers)
    n_rounds += n_rounds % 2

    run_sdf = _sc_phase(True, n, mesh, num_cores, num_workers,
                        n_chunks, n_rounds)
    run_ren = _sc_phase(False, n, mesh, num_cores, num_workers,
                        n_chunks, n_rounds)

    tab = _build_tables(offset, xLayer, yLayer, zLayer).reshape(-1)
    rT = renderPointList.T
    n0, n1, n2 = run_ren(rT[0], rT[1], rT[2], renderIndexList, tab)
    sT = sdfPointList.T
    (sdf_list,) = run_sdf(sT[0], sT[1], sT[2], sdfIndexList, tab)
    return (sdf_list, jnp.stack([n0, n1, n2], axis=1))


# speedup vs baseline: 44.9928x; 1.0006x over previous
import functools

import jax
import jax.numpy as jnp
from jax import lax
from jax.experimental import pallas as pl
from jax.experimental.pallas import tpu as pltpu
from jax.experimental.pallas import tpu_sc as plsc

RESO = 128
CHUNK = 4000
LANES = 16
UNROLL = 5
TAB_ROWS = 16


def _table_body(off_ref, x_ref, y_ref, z_ref, tab_ref):
    f32 = jnp.float32
    row = lax.broadcasted_iota(jnp.int32, (RESO, RESO), 0)
    col = lax.broadcasted_iota(jnp.int32, (RESO, RESO), 1)
    cum = (row <= col).astype(f32)
    shf = (row == col - 1).astype(f32)
    kpos = (lax.broadcasted_iota(jnp.int32, (1, RESO), 1) > 0).astype(f32)
    oneh0 = 1.0 - kpos

    def dot(a, b):
        return jnp.dot(a, b, preferred_element_type=f32)

    for t, l_ref in enumerate((x_ref, y_ref, z_ref)):
        l = l_ref[...]
        off_t = off_ref[t]
        l_m1 = dot(l, shf)
        d = 2.0 * (l_m1 + l * kpos) + off_t * oneh0
        a = dot(d, cum)
        a_m1 = dot(a, shf)
        e = 3.0 * l_m1 + l * kpos + 2.0 * a_m1
        a6 = dot(e, cum)
        if t == 0:
            a6 = a6 + off_ref[3]
        tab_ref[pl.ds(t, 1), :] = l
        tab_ref[pl.ds(3 + t, 1), :] = a
        tab_ref[pl.ds(6 + t, 1), :] = a6
    tab_ref[pl.ds(9, TAB_ROWS - 9), :] = jnp.zeros(
        (TAB_ROWS - 9, RESO), f32)


def _build_tables(offset, x_l, y_l, z_l):
    return pl.pallas_call(
        _table_body,
        out_shape=jax.ShapeDtypeStruct((TAB_ROWS, RESO), jnp.float32),
        in_specs=[
            pl.BlockSpec(memory_space=pltpu.SMEM),
            pl.BlockSpec(),
            pl.BlockSpec(),
            pl.BlockSpec(),
        ],
        out_specs=pl.BlockSpec(),
    )(offset, x_l.reshape(1, RESO), y_l.reshape(1, RESO),
      z_l.reshape(1, RESO))


def _rsqrt(x):
    i = lax.bitcast_convert_type(x, jnp.int32)
    y = lax.bitcast_convert_type(jnp.int32(0x5F3759DF) - (i >> 1), jnp.float32)
    xh = 0.5 * x
    for _ in range(2):
        y = y * (1.5 - xh * y * y)
    return y


def _full(v):
    return jnp.full((LANES,), v, jnp.int32)


def _sc_phase(sdf_mode, n, mesh, num_cores, num_workers, n_chunks, n_rounds):
    n_out = 1 if sdf_mode else 3

    @functools.partial(
        pl.kernel,
        out_type=tuple(jax.ShapeDtypeStruct((n,), jnp.float32)
                       for _ in range(n_out)),
        mesh=mesh,
        compiler_params=pltpu.CompilerParams(needs_layout_passes=False),
        scratch_types=(
            [pltpu.VMEM((TAB_ROWS * RESO,), jnp.float32)]
            + [pltpu.VMEM((CHUNK,), jnp.int32) if k % 4 == 0 else
               pltpu.VMEM((CHUNK,), jnp.float32) for k in range(8)]
            + [pltpu.VMEM((CHUNK,), jnp.float32) for _ in range(2 * n_out)]
            + [pltpu.SemaphoreType.DMA for _ in range(4)]
        ),
    )
    def run(px_h, py_h, pz_h, idx_h, tab_hbm, *rest):
        outs = rest[:n_out]
        tab = rest[n_out]
        bufs = rest[n_out + 1:n_out + 9]
        obufs = rest[n_out + 9:n_out + 9 + 2 * n_out]
        in_sems = rest[-4:-2]
        out_sems = rest[-2:]
        pltpu.sync_copy(tab_hbm, tab)
        in_bufs = (bufs[0:4], bufs[4:8])
        out_bufs = (obufs[0:n_out], obufs[n_out:])

        wid = lax.axis_index("s") * num_cores + lax.axis_index("c")

        def chunk_j(r):
            return wid + r * num_workers

        def in_srcs(sl):
            return (idx_h.at[sl], px_h.at[sl], py_h.at[sl], pz_h.at[sl])

        def out_pairs(j, b):
            sl = pl.ds(j * CHUNK, CHUNK)
            return tuple((o, outs[k].at[sl]) for k, o in enumerate(out_bufs[b]))

        def fire_ins(r, b):
            j = chunk_j(r)

            @pl.when(j < n_chunks)
            def _():
                sl = pl.ds(j * CHUNK, CHUNK)
                for s_ref, d_ref in zip(in_srcs(sl), in_bufs[b]):
                    pltpu.async_copy(s_ref, d_ref, in_sems[b])

        def wait_ins(r, b):
            j = chunk_j(r)

            @pl.when(j < n_chunks)
            def _():
                sl = pl.ds(j * CHUNK, CHUNK)
                for s_ref, d_ref in zip(in_srcs(sl), in_bufs[b]):
                    pltpu.make_async_copy(s_ref, d_ref, in_sems[b]).wait()

        def wait_outs(r, b):
            j = chunk_j(r)

            @pl.when((j >= 0) & (j < n_chunks))
            def _():
                for s_ref, d_ref in out_pairs(j, b):
                    pltpu.make_async_copy(s_ref, d_ref, out_sems[b]).wait()

        def compute(r, b):
            j = chunk_j(r)

            @pl.when(j < n_chunks)
            def _():
                sl = pl.ds(j * CHUNK, CHUNK)
                bi, bx, by, bz = in_bufs[b]
                ob = out_bufs[b]

                def decode(s):
                    idxv = bi[pl.ds(s, 16)]
                    xv = idxv & (RESO - 1)
                    yv = ((idxv >> 7) & (RESO - 1)) + 1 * RESO
                    zv = (idxv >> 14) + 2 * RESO
                    return (xv, yv, zv, bx[pl.ds(s, 16)],
                            by[pl.ds(s, 16)], bz[pl.ds(s, 16)])

                def sdf_vec(s):
                    xv, yv, zv, px, py, pz = decode(s)
                    a0 = plsc.load_gather(tab, [xv]) * px
                    a1 = plsc.load_gather(tab, [yv]) * py
                    a2 = plsc.load_gather(tab, [zv]) * pz
                    a3 = plsc.load_gather(tab, [xv + 3 * RESO])
                    a4 = plsc.load_gather(tab, [yv + 3 * RESO])
                    a5 = plsc.load_gather(tab, [zv + 3 * RESO])
                    a6 = (plsc.load_gather(tab, [xv + 6 * RESO])
                          + plsc.load_gather(tab, [yv + 6 * RESO])
                          + plsc.load_gather(tab, [zv + 6 * RESO]))
                    num = ((a0 + a3) * px + (a1 + a4) * py
                           + (a2 + a5) * pz + a6)
                    u = 2.0 * a0 + a3
                    v = 2.0 * a1 + a4
                    w = 2.0 * a2 + a5
                    ob[0][pl.ds(s, 16)] = (num * _rsqrt(u * u + v * v + w * w)
                                           * (1.0 / RESO))

                def ren_vec(s):
                    xv, yv, zv, px, py, pz = decode(s)
                    g0 = 2.0 * plsc.load_gather(tab, [xv]) * px \
                        + plsc.load_gather(tab, [xv + 3 * RESO])
                    g1 = 2.0 * plsc.load_gather(tab, [yv]) * py \
                        + plsc.load_gather(tab, [yv + 3 * RESO])
                    g2 = 2.0 * plsc.load_gather(tab, [zv]) * pz \
                        + plsc.load_gather(tab, [zv + 3 * RESO])
                    rs = _rsqrt(jnp.maximum(g0 * g0 + g1 * g1 + g2 * g2,
                                            1e-24))
                    ob[0][pl.ds(s, 16)] = g0 * rs
                    ob[1][pl.ds(s, 16)] = g1 * rs
                    ob[2][pl.ds(s, 16)] = g2 * rs

                body = sdf_vec if sdf_mode else ren_vec
                plsc.parallel_loop(0, CHUNK, LANES, unroll=UNROLL)(body)
                for o, d in out_pairs(j, b):
                    pltpu.async_copy(o, d, out_sems[b])

        def step(r, b):
            fire_ins(r + 1, 1 - b)
            wait_ins(r, b)
            wait_outs(r - 2, b)
            compute(r, b)

        fire_ins(jnp.int32(0), 0)

        def pair_body(rr, carry_):
            r = 2 * rr
            step(r, 0)
            step(r + 1, 1)
            return carry_

        lax.fori_loop(0, n_rounds // 2, pair_body, 0)
        wait_outs(jnp.int32(n_rounds - 2), 0)
        wait_outs(jnp.int32(n_rounds - 1), 1)

    return run


def kernel(renderPointList, renderIndexList, sdfPointList, sdfIndexList,
           xLayer, yLayer, zLayer, offset):
    n = sdfPointList.shape[0]
    assert n % CHUNK == 0
    n_chunks = n // CHUNK

    mesh = plsc.VectorSubcoreMesh(core_axis_name="c", subcore_axis_name="s")
    num_cores = mesh.num_cores
    num_workers = num_cores * mesh.num_subcores
    n_rounds = -(-n_chunks // num_workers)
    n_rounds += n_rounds % 2

    run_sdf = _sc_phase(True, n, mesh, num_cores, num_workers,
                        n_chunks, n_rounds)
    run_ren = _sc_phase(False, n, mesh, num_cores, num_workers,
                        n_chunks, n_rounds)

    tab = _build_tables(offset, xLayer, yLayer, zLayer).reshape(-1)
    rT = renderPointList.T
    n0, n1, n2 = run_ren(rT[0], rT[1], rT[2], renderIndexList, tab)
    sT = sdfPointList.T
    (sdf_list,) = run_sdf(sT[0], sT[1], sT[2], sdfIndexList, tab)
    return (sdf_list, jnp.stack([n0, n1, n2], axis=1))
